# Initial kernel scaffold; baseline (speedup 1.0000x reference)
import jax
import jax.numpy as jnp
from jax.experimental import pallas as pl


def kernel(query, key, value):
    raise NotImplementedError("write your pallas kernel here")


# speedup vs baseline: 1.1000x; 1.1000x over previous
import functools
import math

import jax
import jax.numpy as jnp
from jax.experimental import pallas as pl

DIM = 128
KV_LEN = 8192


def _attn_kernel(q_ref, k_ref, v_ref, ctx_ref, attn_ref):
    q = q_ref[0]
    k = k_ref[0]
    v = v_ref[0]
    s = jax.lax.dot_general(
        q, k, (((1,), (1,)), ((), ())),
        preferred_element_type=jnp.float32,
    ) * (1.0 / math.sqrt(DIM))
    m = jnp.max(s, axis=-1, keepdims=True)
    p = jnp.exp(s - m)
    denom = jnp.sum(p, axis=-1, keepdims=True)
    attn = p / denom
    ctx = jnp.dot(attn, v, preferred_element_type=jnp.float32)
    attn_ref[0] = attn
    ctx_ref[0] = ctx


@jax.jit
def kernel(query, key, value):
    batch, q_len, dim = query.shape
    kv_len = key.shape[1]
    grid = (batch,)
    out_ctx = jax.ShapeDtypeStruct((batch, q_len, dim), jnp.float32)
    out_attn = jax.ShapeDtypeStruct((batch, q_len, kv_len), jnp.float32)
    ctx, attn = pl.pallas_call(
        _attn_kernel,
        grid=grid,
        in_specs=[
            pl.BlockSpec((1, q_len, dim), lambda b: (b, 0, 0)),
            pl.BlockSpec((1, kv_len, dim), lambda b: (b, 0, 0)),
            pl.BlockSpec((1, kv_len, dim), lambda b: (b, 0, 0)),
        ],
        out_specs=[
            pl.BlockSpec((1, q_len, dim), lambda b: (b, 0, 0)),
            pl.BlockSpec((1, q_len, kv_len), lambda b: (b, 0, 0)),
        ],
        out_shape=[out_ctx, out_attn],
    )(query, key, value)
    return (ctx, attn)


# speedup vs baseline: 1.1007x; 1.0006x over previous
import functools
import math

import jax
import jax.numpy as jnp
from jax.experimental import pallas as pl
from jax.experimental.pallas import tpu as pltpu

DIM = 128
KV_LEN = 8192


def _attn_kernel(q_ref, k_ref, v_ref, ctx_ref, attn_ref):
    q = q_ref[0]
    k = k_ref[0]
    v = v_ref[0]
    s = jax.lax.dot_general(
        q, k, (((1,), (1,)), ((), ())),
        preferred_element_type=jnp.float32,
    ) * (1.0 / math.sqrt(DIM))
    m = jnp.max(s, axis=-1, keepdims=True)
    p = jnp.exp(s - m)
    denom = jnp.sum(p, axis=-1, keepdims=True)
    attn = p / denom
    ctx = jnp.dot(attn, v, preferred_element_type=jnp.float32)
    attn_ref[0] = attn
    ctx_ref[0] = ctx


@jax.jit
def kernel(query, key, value):
    batch, q_len, dim = query.shape
    kv_len = key.shape[1]
    grid = (batch,)
    out_ctx = jax.ShapeDtypeStruct((batch, q_len, dim), jnp.float32)
    out_attn = jax.ShapeDtypeStruct((batch, q_len, kv_len), jnp.float32)
    ctx, attn = pl.pallas_call(
        _attn_kernel,
        grid=grid,
        in_specs=[
            pl.BlockSpec((1, q_len, dim), lambda b: (b, 0, 0)),
            pl.BlockSpec((1, kv_len, dim), lambda b: (b, 0, 0)),
            pl.BlockSpec((1, kv_len, dim), lambda b: (b, 0, 0)),
        ],
        out_specs=[
            pl.BlockSpec((1, q_len, dim), lambda b: (b, 0, 0)),
            pl.BlockSpec((1, q_len, kv_len), lambda b: (b, 0, 0)),
        ],
        out_shape=[out_ctx, out_attn],
        compiler_params=pltpu.CompilerParams(
            dimension_semantics=("parallel",),
        ),
    )(query, key, value)
    return (ctx, attn)


# speedup vs baseline: 1.1212x; 1.0186x over previous
import functools
import math

import jax
import jax.numpy as jnp
from jax.experimental import pallas as pl
from jax.experimental.pallas import tpu as pltpu

DIM = 128
KV_LEN = 8192


BB = 2


def _attn_kernel(q_ref, k_ref, v_ref, ctx_ref, attn_ref):
    for i in range(BB):
        q = q_ref[i]
        k = k_ref[i]
        v = v_ref[i]
        s = jax.lax.dot_general(
            q, k, (((1,), (1,)), ((), ())),
            preferred_element_type=jnp.float32,
        ) * (1.0 / math.sqrt(DIM))
        m = jnp.max(s, axis=-1, keepdims=True)
        p = jnp.exp(s - m)
        denom = jnp.sum(p, axis=-1, keepdims=True)
        attn = p / denom
        ctx = jnp.dot(attn, v, preferred_element_type=jnp.float32)
        attn_ref[i] = attn
        ctx_ref[i] = ctx


@jax.jit
def kernel(query, key, value):
    batch, q_len, dim = query.shape
    kv_len = key.shape[1]
    grid = (batch // BB,)
    out_ctx = jax.ShapeDtypeStruct((batch, q_len, dim), jnp.float32)
    out_attn = jax.ShapeDtypeStruct((batch, q_len, kv_len), jnp.float32)
    ctx, attn = pl.pallas_call(
        _attn_kernel,
        grid=grid,
        in_specs=[
            pl.BlockSpec((BB, q_len, dim), lambda b: (b, 0, 0)),
            pl.BlockSpec((BB, kv_len, dim), lambda b: (b, 0, 0)),
            pl.BlockSpec((BB, kv_len, dim), lambda b: (b, 0, 0)),
        ],
        out_specs=[
            pl.BlockSpec((BB, q_len, dim), lambda b: (b, 0, 0)),
            pl.BlockSpec((BB, q_len, kv_len), lambda b: (b, 0, 0)),
        ],
        out_shape=[out_ctx, out_attn],
        compiler_params=pltpu.CompilerParams(
            dimension_semantics=("parallel",),
        ),
    )(query, key, value)
    return (ctx, attn)


# speedup vs baseline: 1.1237x; 1.0022x over previous
import functools
import math

import jax
import jax.numpy as jnp
from jax.experimental import pallas as pl
from jax.experimental.pallas import tpu as pltpu

DIM = 128
KV_LEN = 8192
BB = 2
HALF = KV_LEN // 2


def _attn_kernel(q_ref, k1_ref, k2_ref, v1_ref, v2_ref, ctx_ref, attn_ref):
    scale = 1.0 / math.sqrt(DIM)
    for i in range(BB):
        q = q_ref[i]
        s1 = jax.lax.dot_general(
            q, k1_ref[i], (((1,), (1,)), ((), ())),
            preferred_element_type=jnp.float32,
        ) * scale
        s2 = jax.lax.dot_general(
            q, k2_ref[i], (((1,), (1,)), ((), ())),
            preferred_element_type=jnp.float32,
        ) * scale
        m = jnp.maximum(jnp.max(s1), jnp.max(s2))
        p1 = jnp.exp(s1 - m)
        p2 = jnp.exp(s2 - m)
        denom = jnp.sum(p1) + jnp.sum(p2)
        inv = 1.0 / denom
        a1 = p1 * inv
        a2 = p2 * inv
        ctx = (
            jnp.dot(a1, v1_ref[i], preferred_element_type=jnp.float32)
            + jnp.dot(a2, v2_ref[i], preferred_element_type=jnp.float32)
        )
        attn_ref[i, :, :HALF] = a1
        attn_ref[i, :, HALF:] = a2
        ctx_ref[i] = ctx


@jax.jit
def kernel(query, key, value):
    batch, q_len, dim = query.shape
    kv_len = key.shape[1]
    half = kv_len // 2
    grid = (batch // BB,)
    out_ctx = jax.ShapeDtypeStruct((batch, q_len, dim), jnp.float32)
    out_attn = jax.ShapeDtypeStruct((batch, q_len, kv_len), jnp.float32)
    ctx, attn = pl.pallas_call(
        _attn_kernel,
        grid=grid,
        in_specs=[
            pl.BlockSpec((BB, q_len, dim), lambda b: (b, 0, 0)),
            pl.BlockSpec((BB, half, dim), lambda b: (b, 0, 0)),
            pl.BlockSpec((BB, half, dim), lambda b: (b, 1, 0)),
            pl.BlockSpec((BB, half, dim), lambda b: (b, 0, 0)),
            pl.BlockSpec((BB, half, dim), lambda b: (b, 1, 0)),
        ],
        out_specs=[
            pl.BlockSpec((BB, q_len, dim), lambda b: (b, 0, 0)),
            pl.BlockSpec((BB, q_len, kv_len), lambda b: (b, 0, 0)),
        ],
        out_shape=[out_ctx, out_attn],
        compiler_params=pltpu.CompilerParams(
            dimension_semantics=("parallel",),
        ),
    )(query, key, key, value, value)
    return (ctx, attn)
